# baseline (speedup 1.0000x reference)
import functools

import jax
import jax.numpy as jnp
from jax import lax
from jax.experimental import pallas as pl
from jax.experimental.pallas import tpu as pltpu
from jax.experimental.pallas import tpu_sc as plsc

B = 16384
S = 13
E = 16
C = 2


def kernel(x, emb_table, fc_w, fc_b):
    info = plsc.get_sparse_core_info()
    nw = info.num_cores * info.num_subcores
    bw = B // nw
    n_idx = bw * S

    x_flat = x.reshape(-1).astype(jnp.int32)
    w = fc_w.reshape(-1)
    b_pad = jnp.zeros((E,), jnp.float32).at[:C].set(fc_b)
    eye = lax.optimization_barrier(jnp.eye(E, dtype=jnp.float32))
    tbl_lin = emb_table @ eye

    mesh = plsc.VectorSubcoreMesh(core_axis_name="c", subcore_axis_name="s")

    @functools.partial(
        pl.kernel,
        mesh=mesh,
        out_type=jax.ShapeDtypeStruct((B * C,), jnp.float32),
        compiler_params=pltpu.CompilerParams(
            needs_layout_passes=False, use_tc_tiling_on_sc=False),
        scratch_types=[
            pltpu.VMEM((n_idx,), jnp.int32),
            pltpu.VMEM((n_idx, E), jnp.float32),
            pltpu.VMEM((C * S * E,), jnp.float32),
            pltpu.VMEM((E,), jnp.float32),
            pltpu.VMEM((bw * C,), jnp.float32),
            pltpu.SemaphoreType.DMA,
        ],
    )
    def sc_kernel(x_hbm, tbl_hbm, w_hbm, b_hbm, out_hbm,
                  idx_v, rows_v, w_v, b_v, out_v, sem):
        wid = lax.axis_index("s") * info.num_cores + lax.axis_index("c")
        base = wid * bw
        pltpu.sync_copy(x_hbm.at[pl.ds(base * S, n_idx)], idx_v)
        pltpu.sync_copy(w_hbm, w_v)
        pltpu.sync_copy(b_hbm, b_v)
        pltpu.async_copy(tbl_hbm.at[idx_v], rows_v, sem).wait()

        wv = [[w_v[pl.ds((c * S + s) * E, E)] for s in range(S)] for c in range(C)]
        bvec = b_v[...]
        b0 = bvec[0]
        b1 = bvec[1]
        lanes = lax.iota(jnp.int32, 16)

        def body(g, carry):
            vec = jnp.zeros((16,), jnp.float32)
            for j in range(8):
                o = (g * 8 + j) * S
                r = rows_v[o]
                acc0 = r * wv[0][0]
                acc1 = r * wv[1][0]
                for s in range(1, S):
                    r = rows_v[o + s]
                    acc0 = acc0 + r * wv[0][s]
                    acc1 = acc1 + r * wv[1][s]
                l0 = jnp.sum(acc0) + b0
                l1 = jnp.sum(acc1) + b1
                vec = jnp.where(lanes == 2 * j, l0, vec)
                vec = jnp.where(lanes == 2 * j + 1, l1, vec)
            out_v[pl.ds(g * 16, 16)] = vec
            return carry

        lax.fori_loop(0, bw // 8, body, 0)
        pltpu.sync_copy(out_v, out_hbm.at[pl.ds(base * C, bw * C)])

    out = sc_kernel(x_flat, tbl_lin, w, b_pad)
    return out.reshape(B, C)


# speedup vs baseline: 2.2211x; 2.2211x over previous
import functools

import jax
import jax.numpy as jnp
from jax import lax
from jax.experimental import pallas as pl
from jax.experimental.pallas import tpu as pltpu
from jax.experimental.pallas import tpu_sc as plsc

B = 16384
S = 13
E = 16
C = 2


def kernel(x, emb_table, fc_w, fc_b):
    info = plsc.get_sparse_core_info()
    nw = info.num_cores * info.num_subcores
    bw = B // nw
    n_idx = bw * S

    x_flat = x.reshape(-1).astype(jnp.int32) * 8
    w = fc_w.reshape(-1)
    b_pad = jnp.zeros((E,), jnp.float32).at[:C].set(fc_b)
    proj = jnp.concatenate(
        [jnp.eye(E, dtype=jnp.float32),
         jnp.zeros((E, 128 - E), jnp.float32)], axis=1)
    proj = lax.optimization_barrier(proj)
    tbl_lin = (emb_table @ proj).reshape(emb_table.shape[0] * 8, E)

    mesh = plsc.VectorSubcoreMesh(core_axis_name="c", subcore_axis_name="s")

    @functools.partial(
        pl.kernel,
        mesh=mesh,
        out_type=jax.ShapeDtypeStruct((B * C,), jnp.float32),
        compiler_params=pltpu.CompilerParams(
            needs_layout_passes=False, use_tc_tiling_on_sc=False),
        scratch_types=[
            pltpu.VMEM((n_idx,), jnp.int32),
            pltpu.VMEM((n_idx, E), jnp.float32),
            pltpu.VMEM((C * S * E,), jnp.float32),
            pltpu.VMEM((E,), jnp.float32),
            pltpu.VMEM((bw * C,), jnp.float32),
            pltpu.SemaphoreType.DMA,
        ],
    )
    def sc_kernel(x_hbm, tbl_hbm, w_hbm, b_hbm, out_hbm,
                  idx_v, rows_v, w_v, b_v, out_v, sem):
        wid = lax.axis_index("s") * info.num_cores + lax.axis_index("c")
        base = wid * bw
        pltpu.sync_copy(x_hbm.at[pl.ds(base * S, n_idx)], idx_v)
        pltpu.sync_copy(w_hbm, w_v)
        pltpu.sync_copy(b_hbm, b_v)
        pltpu.async_copy(tbl_hbm.at[idx_v], rows_v, sem).wait()

        wv = [[w_v[pl.ds((c * S + s) * E, E)] for s in range(S)] for c in range(C)]
        bvec = b_v[...]
        b0 = bvec[0]
        b1 = bvec[1]
        lanes = lax.iota(jnp.int32, 16)

        def body(g, carry):
            vec = jnp.zeros((16,), jnp.float32)
            for j in range(8):
                o = (g * 8 + j) * S
                r = rows_v[o]
                acc0 = r * wv[0][0]
                acc1 = r * wv[1][0]
                for s in range(1, S):
                    r = rows_v[o + s]
                    acc0 = acc0 + r * wv[0][s]
                    acc1 = acc1 + r * wv[1][s]
                l0 = jnp.sum(acc0) + b0
                l1 = jnp.sum(acc1) + b1
                vec = jnp.where(lanes == 2 * j, l0, vec)
                vec = jnp.where(lanes == 2 * j + 1, l1, vec)
            out_v[pl.ds(g * 16, 16)] = vec
            return carry

        lax.fori_loop(0, bw // 8, body, 0)
        pltpu.sync_copy(out_v, out_hbm.at[pl.ds(base * C, bw * C)])

    out = sc_kernel(x_flat, tbl_lin, w, b_pad)
    return out.reshape(B, C)
